# Initial kernel scaffold; baseline (speedup 1.0000x reference)
import jax
import jax.numpy as jnp
from jax.experimental import pallas as pl


def kernel(X):
    raise NotImplementedError("write your pallas kernel here")


# speedup vs baseline: 4.6360x; 4.6360x over previous
import functools

import jax
import jax.numpy as jnp
from jax import lax
from jax.experimental import pallas as pl
from jax.experimental.pallas import tpu as pltpu
from jax.experimental.pallas import tpu_sc as plsc

NUM_ROWS = 128
N = 32768
RANK = 257
NC = 2
NS = 16
L = 16
NW = NC * NS
ROWS_PER_W = NUM_ROWS // NW
NV = N // L
NB = 256
ABS_MASK_I = 0x7FFFFFFF
SIGN_MASK_I = -0x80000000
LEVELS = ((23, 8), (15, 8), (7, 8), (0, 7))


def _process_row(x_hbm, out_hbm, row, data, hist, sref):
    ABS_MASK = jnp.int32(ABS_MASK_I)
    SIGN_MASK = jnp.int32(SIGN_MASK_I)
    pltpu.sync_copy(x_hbm.at[row], data)
    lane_off = lax.iota(jnp.int32, L) * NB
    ones = jnp.ones((L,), jnp.int32)

    r = jnp.int32(RANK)
    prefix = jnp.int32(0)
    for s, w in LEVELS:
        nb = 1 << w
        top = s + w

        def zero_body(j, _):
            hist[pl.ds(j * L, L)] = jnp.zeros((L,), jnp.int32)
            return 0
        lax.fori_loop(0, (L * NB) // L, zero_body, 0)

        def hist_body(j, _, s=s, top=top, nb=nb):
            x = data[pl.ds(j * L, L)]
            u = plsc.bitcast(x, jnp.int32) & ABS_MASK
            b = lax.shift_right_logical(u, jnp.int32(s)) & jnp.int32(nb - 1)
            idx = b + lane_off
            if top >= 31:
                plsc.addupdate_scatter(hist, [idx], ones)
            else:
                m = lax.shift_right_logical(u, jnp.int32(top)) == prefix
                plsc.addupdate_scatter(hist, [idx], ones, mask=m)
            return 0
        lax.fori_loop(0, NV, hist_body, 0)

        sref[pl.ds(nb, L)] = jnp.zeros((L,), jnp.int32)

        def scan_body(jj, carry, nb=nb):
            run, npos = carry
            j = (nb // L - 1) - jj
            tot = hist[pl.ds(j * L, L)]
            for lane in range(1, L):
                tot = tot + hist[pl.ds(lane * NB + j * L, L)]
            schunk = lax.rev(plsc.cumsum(lax.rev(tot, (0,))), (0,)) + run
            sref[pl.ds(j * L, L)] = schunk
            npos = npos + plsc.all_reduce_population_count(schunk >= r)
            return jnp.max(schunk), npos
        _, nposv = lax.fori_loop(
            0, nb // L, scan_body,
            (jnp.int32(0), jnp.zeros((L,), jnp.int32)))

        bstar = jnp.max(nposv) - 1
        s_above = jnp.max(
            plsc.load_gather(sref, [jnp.full((L,), bstar + 1, jnp.int32)]))
        r = r - s_above
        prefix = prefix * jnp.int32(nb) + bstar

    t = prefix

    def min_gt_body(j, acc):
        x = data[pl.ds(j * L, L)]
        u = plsc.bitcast(x, jnp.int32) & ABS_MASK
        return jnp.minimum(acc, jnp.where(u > t, u, ABS_MASK))
    accv = lax.fori_loop(0, NV, min_gt_body, jnp.full((L,), ABS_MASK))
    v256b = jnp.where(r == jnp.int32(1), jnp.min(accv), t)

    v257f = plsc.bitcast(jnp.full((L,), t, jnp.int32), jnp.float32)
    v256f = plsc.bitcast(jnp.full((L,), v256b, jnp.int32), jnp.float32)
    qv = v257f + jnp.float32(0.0078125) * (v256f - v257f)

    def out_body(j, _):
        x = data[pl.ds(j * L, L)]
        bits = plsc.bitcast(x, jnp.int32)
        af = plsc.bitcast(bits & ABS_MASK, jnp.float32)
        d = jnp.maximum(af - qv, jnp.float32(0.0))
        o = plsc.bitcast(d, jnp.int32) | (bits & SIGN_MASK)
        data[pl.ds(j * L, L)] = plsc.bitcast(o, jnp.float32)
        return 0
    lax.fori_loop(0, NV, out_body, 0)
    pltpu.sync_copy(data, out_hbm.at[row])


def _sc_call(x):
    mesh = plsc.VectorSubcoreMesh(
        core_axis_name="c", subcore_axis_name="s",
        num_cores=NC, num_subcores=NS)

    @functools.partial(
        pl.kernel,
        out_type=jax.ShapeDtypeStruct((NUM_ROWS, N), jnp.float32),
        mesh=mesh,
        compiler_params=pltpu.CompilerParams(needs_layout_passes=False),
        scratch_types=[
            pltpu.VMEM((N,), jnp.float32),
            pltpu.VMEM((L * NB,), jnp.int32),
            pltpu.VMEM((NB + L,), jnp.int32),
        ],
    )
    def k(x_hbm, out_hbm, data, hist, sref):
        wid = lax.axis_index("s") * NC + lax.axis_index("c")
        for i in range(ROWS_PER_W):
            _process_row(x_hbm, out_hbm, wid * ROWS_PER_W + i,
                         data, hist, sref)

    return k(x)


def kernel(X):
    return _sc_call(X)


# speedup vs baseline: 19.3173x; 4.1668x over previous
import functools

import jax
import jax.numpy as jnp
from jax import lax
from jax.experimental import pallas as pl
from jax.experimental.pallas import tpu as pltpu
from jax.experimental.pallas import tpu_sc as plsc

NUM_ROWS = 128
N = 32768
RANK = 257
NC = 2
NS = 16
L = 16
NW = NC * NS
ROWS_PER_W = NUM_ROWS // NW
NV = N // L
NB = 256
ABS_MASK_I = 0x7FFFFFFF
SIGN_MASK_I = -0x80000000
UNROLL = 8


def _scan_level(hist, sref, r, nb):
    sref[pl.ds(nb, L)] = jnp.zeros((L,), jnp.int32)

    def scan_body(jj, carry):
        run, npos = carry
        j = (nb // L - 1) - jj
        tot = hist[pl.ds(j * L, L)]
        for lane in range(1, L):
            tot = tot + hist[pl.ds(lane * NB + j * L, L)]
        schunk = lax.rev(plsc.cumsum(lax.rev(tot, (0,))), (0,)) + run
        sref[pl.ds(j * L, L)] = schunk
        npos = npos + plsc.all_reduce_population_count(schunk >= r)
        return jnp.max(schunk), npos

    _, nposv = lax.fori_loop(
        0, nb // L, scan_body, (jnp.int32(0), jnp.zeros((L,), jnp.int32)))
    bstar = jnp.max(nposv) - 1
    s_above = jnp.max(
        plsc.load_gather(sref, [jnp.full((L,), bstar + 1, jnp.int32)]))
    return bstar, s_above


def _zero_hist(hist):
    @plsc.parallel_loop(0, (L * NB) // L, unroll=UNROLL)
    def _(j):
        hist[pl.ds(j * L, L)] = jnp.zeros((L,), jnp.int32)


def _process_row(x_hbm, out_hbm, row, data, cand, hist, sref):
    ABS_MASK = jnp.int32(ABS_MASK_I)
    SIGN_MASK = jnp.int32(SIGN_MASK_I)
    pltpu.sync_copy(x_hbm.at[row], data)
    lanes = lax.iota(jnp.int32, L)
    lane_off = lanes * NB
    ones = jnp.ones((L,), jnp.int32)

    _zero_hist(hist)

    @plsc.parallel_loop(0, NV, unroll=UNROLL)
    def _(j):
        x = data[pl.ds(j * L, L)]
        u = plsc.bitcast(x, jnp.int32) & ABS_MASK
        plsc.addupdate_scatter(
            hist, [lax.shift_right_logical(u, jnp.int32(23)) + lane_off], ones)

    b0, s_above = _scan_level(hist, sref, jnp.int32(RANK), NB)
    r = jnp.int32(RANK) - s_above

    _zero_hist(hist)

    @plsc.parallel_loop(0, NV, unroll=UNROLL, carry=jnp.zeros((L,), jnp.int32))
    def off_vec(j, off):
        x = data[pl.ds(j * L, L)]
        u = plsc.bitcast(x, jnp.int32) & ABS_MASK
        e = lax.shift_right_logical(u, jnp.int32(23))
        b = lax.shift_right_logical(u, jnp.int32(15)) & jnp.int32(0xFF)
        plsc.addupdate_scatter(hist, [b + lane_off], ones, mask=e == b0)
        m_ge = e >= b0
        plsc.store_scatter(cand, [off * L + lanes], u, mask=m_ge)
        return off + jnp.where(m_ge, 1, 0)

    b1, s_above = _scan_level(hist, sref, r, NB)
    r = r - s_above
    prefix = b0 * jnp.int32(NB) + b1
    jmax = jnp.max(off_vec)

    _zero_hist(hist)

    @plsc.parallel_loop(0, jmax, unroll=2)
    def _(j):
        u = cand[pl.ds(j * L, L)]
        b = lax.shift_right_logical(u, jnp.int32(7)) & jnp.int32(0xFF)
        m = (off_vec > j) & (lax.shift_right_logical(u, jnp.int32(15)) == prefix)
        plsc.addupdate_scatter(hist, [b + lane_off], ones, mask=m)

    b2, s_above = _scan_level(hist, sref, r, NB)
    r = r - s_above
    prefix = prefix * jnp.int32(NB) + b2

    _zero_hist(hist)

    @plsc.parallel_loop(0, jmax, unroll=2)
    def _(j):
        u = cand[pl.ds(j * L, L)]
        b = u & jnp.int32(0x7F)
        m = (off_vec > j) & (lax.shift_right_logical(u, jnp.int32(7)) == prefix)
        plsc.addupdate_scatter(hist, [b + lane_off], ones, mask=m)

    b3, s_above = _scan_level(hist, sref, r, 128)
    r = r - s_above
    t = prefix * jnp.int32(128) + b3

    @plsc.parallel_loop(0, jmax, unroll=2, carry=jnp.full((L,), ABS_MASK))
    def accv(j, acc):
        u = cand[pl.ds(j * L, L)]
        return jnp.minimum(acc, jnp.where((off_vec > j) & (u > t), u, ABS_MASK))

    v256b = jnp.where(r == jnp.int32(1), jnp.min(accv), t)

    v257f = plsc.bitcast(jnp.full((L,), t, jnp.int32), jnp.float32)
    v256f = plsc.bitcast(jnp.full((L,), v256b, jnp.int32), jnp.float32)
    qv = v257f + jnp.float32(0.0078125) * (v256f - v257f)

    @plsc.parallel_loop(0, NV, unroll=UNROLL)
    def _(j):
        x = data[pl.ds(j * L, L)]
        bits = plsc.bitcast(x, jnp.int32)
        af = plsc.bitcast(bits & ABS_MASK, jnp.float32)
        d = jnp.maximum(af - qv, jnp.float32(0.0))
        o = plsc.bitcast(d, jnp.int32) | (bits & SIGN_MASK)
        data[pl.ds(j * L, L)] = plsc.bitcast(o, jnp.float32)

    pltpu.sync_copy(data, out_hbm.at[row])


def _sc_call(x):
    mesh = plsc.VectorSubcoreMesh(
        core_axis_name="c", subcore_axis_name="s",
        num_cores=NC, num_subcores=NS)

    @functools.partial(
        pl.kernel,
        out_type=jax.ShapeDtypeStruct((NUM_ROWS, N), jnp.float32),
        mesh=mesh,
        compiler_params=pltpu.CompilerParams(needs_layout_passes=False),
        scratch_types=[
            pltpu.VMEM((N,), jnp.float32),
            pltpu.VMEM((N,), jnp.int32),
            pltpu.VMEM((L * NB,), jnp.int32),
            pltpu.VMEM((NB + L,), jnp.int32),
        ],
    )
    def k(x_hbm, out_hbm, data, cand, hist, sref):
        wid = lax.axis_index("s") * NC + lax.axis_index("c")
        for i in range(ROWS_PER_W):
            _process_row(x_hbm, out_hbm, wid * ROWS_PER_W + i,
                         data, cand, hist, sref)

    return k(x)


def kernel(X):
    return _sc_call(X)


# speedup vs baseline: 22.8335x; 1.1820x over previous
import functools

import jax
import jax.numpy as jnp
from jax import lax
from jax.experimental import pallas as pl
from jax.experimental.pallas import tpu as pltpu
from jax.experimental.pallas import tpu_sc as plsc

NUM_ROWS = 128
N = 32768
RANK = 257
NC = 2
NS = 16
L = 16
NW = NC * NS
ROWS_PER_W = NUM_ROWS // NW
NV = N // L
NB = 256
ABS_MASK_I = 0x7FFFFFFF
SIGN_MASK_I = -0x80000000
UNROLL = 8


def _scan_level(hist, sref, r, nb):
    sref[pl.ds(nb, L)] = jnp.zeros((L,), jnp.int32)

    def scan_body(jj, carry):
        run, npos = carry
        j = (nb // L - 1) - jj
        tot = hist[pl.ds(j * L, L)]
        schunk = lax.rev(plsc.cumsum(lax.rev(tot, (0,))), (0,)) + run
        sref[pl.ds(j * L, L)] = schunk
        npos = npos + plsc.all_reduce_population_count(schunk >= r)
        return jnp.max(schunk), npos

    _, nposv = lax.fori_loop(
        0, nb // L, scan_body, (jnp.int32(0), jnp.zeros((L,), jnp.int32)))
    bstar = jnp.max(nposv) - 1
    s_above = jnp.max(
        plsc.load_gather(sref, [jnp.full((L,), bstar + 1, jnp.int32)]))
    return bstar, s_above


def _zero_hist(hist):
    for j in range(NB // L):
        hist[pl.ds(j * L, L)] = jnp.zeros((L,), jnp.int32)


def _row_quantile(data, cand, hist, sref):
    ABS_MASK = jnp.int32(ABS_MASK_I)
    lanes = lax.iota(jnp.int32, L)
    ones = jnp.ones((L,), jnp.int32)

    _zero_hist(hist)

    @plsc.parallel_loop(0, NV, unroll=UNROLL)
    def _(j):
        x = data[pl.ds(j * L, L)]
        b = lax.shift_right_logical(
            plsc.bitcast(x, jnp.int32), jnp.int32(23)) & jnp.int32(0xFF)
        plsc.addupdate_scatter(hist, [b], ones)

    b0, s_above = _scan_level(hist, sref, jnp.int32(RANK), NB)
    r = jnp.int32(RANK) - s_above

    @plsc.parallel_loop(0, NV, unroll=UNROLL, carry=jnp.zeros((L,), jnp.int32))
    def off_vec(j, off):
        x = data[pl.ds(j * L, L)]
        u = plsc.bitcast(x, jnp.int32) & ABS_MASK
        m_ge = lax.shift_right_logical(u, jnp.int32(23)) >= b0
        plsc.store_scatter(cand, [off * L + lanes], u, mask=m_ge)
        return off + jnp.where(m_ge, 1, 0)

    jmax = jnp.max(off_vec)

    prefix = b0
    for shift, width in ((15, 8), (7, 8), (0, 7)):
        nb = 1 << width
        _zero_hist(hist)

        @plsc.parallel_loop(0, jmax, unroll=2)
        def _(j, shift=shift, nb=nb, prefix=prefix, r=r):
            u = cand[pl.ds(j * L, L)]
            b = lax.shift_right_logical(u, jnp.int32(shift)) & jnp.int32(nb - 1)
            m = (off_vec > j) & (
                lax.shift_right_logical(u, jnp.int32(shift + width)) == prefix)
            plsc.addupdate_scatter(hist, [b], ones, mask=m)

        bl, s_above = _scan_level(hist, sref, r, nb)
        r = r - s_above
        prefix = prefix * jnp.int32(nb) + bl

    t = prefix

    @plsc.parallel_loop(0, jmax, unroll=2, carry=jnp.full((L,), ABS_MASK))
    def accv(j, acc):
        u = cand[pl.ds(j * L, L)]
        return jnp.minimum(acc, jnp.where((off_vec > j) & (u > t), u, ABS_MASK))

    v256b = jnp.where(r == jnp.int32(1), jnp.min(accv), t)

    v257f = plsc.bitcast(jnp.full((L,), t, jnp.int32), jnp.float32)
    v256f = plsc.bitcast(jnp.full((L,), v256b, jnp.int32), jnp.float32)
    return v257f + jnp.float32(0.0078125) * (v256f - v257f)


def _threshold_row(data, qv):
    ABS_MASK = jnp.int32(ABS_MASK_I)
    SIGN_MASK = jnp.int32(SIGN_MASK_I)

    @plsc.parallel_loop(0, NV, unroll=UNROLL)
    def _(j):
        x = data[pl.ds(j * L, L)]
        bits = plsc.bitcast(x, jnp.int32)
        af = plsc.bitcast(bits & ABS_MASK, jnp.float32)
        d = jnp.maximum(af - qv, jnp.float32(0.0))
        o = plsc.bitcast(d, jnp.int32) | (bits & SIGN_MASK)
        data[pl.ds(j * L, L)] = plsc.bitcast(o, jnp.float32)


def _sc_call(x):
    mesh = plsc.VectorSubcoreMesh(
        core_axis_name="c", subcore_axis_name="s",
        num_cores=NC, num_subcores=NS)

    @functools.partial(
        pl.kernel,
        out_type=jax.ShapeDtypeStruct((NUM_ROWS, N), jnp.float32),
        mesh=mesh,
        compiler_params=pltpu.CompilerParams(needs_layout_passes=False),
        scratch_types=[
            pltpu.VMEM((N,), jnp.float32),
            pltpu.VMEM((N,), jnp.float32),
            pltpu.VMEM((N,), jnp.int32),
            pltpu.VMEM((NB + L,), jnp.int32),
            pltpu.VMEM((NB + L,), jnp.int32),
            pltpu.SemaphoreType.DMA,
            pltpu.SemaphoreType.DMA,
            pltpu.SemaphoreType.DMA,
            pltpu.SemaphoreType.DMA,
        ],
    )
    def k(x_hbm, out_hbm, data0, data1, cand, hist, sref,
          lsem0, lsem1, ssem0, ssem1):
        wid = lax.axis_index("s") * NC + lax.axis_index("c")
        base = wid * ROWS_PER_W
        bufs = (data0, data1)
        lsems = (lsem0, lsem1)
        ssems = (ssem0, ssem1)

        loads = [pltpu.async_copy(x_hbm.at[base], data0, lsem0)]
        stores = [None, None]
        for i in range(ROWS_PER_W):
            cur = i % 2
            nxt = (i + 1) % 2
            if i + 1 < ROWS_PER_W:
                if stores[nxt] is not None:
                    stores[nxt].wait()
                    stores[nxt] = None
                loads.append(pltpu.async_copy(
                    x_hbm.at[base + i + 1], bufs[nxt], lsems[nxt]))
            loads[i].wait()
            data = bufs[cur]
            qv = _row_quantile(data, cand, hist, sref)
            _threshold_row(data, qv)
            stores[cur] = pltpu.async_copy(
                data, out_hbm.at[base + i], ssems[cur])
        for s in stores:
            if s is not None:
                s.wait()

    return k(x)


def kernel(X):
    return _sc_call(X)


# speedup vs baseline: 24.5105x; 1.0734x over previous
import functools

import jax
import jax.numpy as jnp
from jax import lax
from jax.experimental import pallas as pl
from jax.experimental.pallas import tpu as pltpu
from jax.experimental.pallas import tpu_sc as plsc

NUM_ROWS = 128
N = 32768
RANK = 257
NC = 2
NS = 16
L = 16
NW = NC * NS
ROWS_PER_W = NUM_ROWS // NW
NV = N // L
NB = 256
ABS_MASK_I = 0x7FFFFFFF
SIGN_MASK_I = -0x80000000
UNROLL = 8


def _scan_level(hist, sref, r, nb):
    sref[pl.ds(nb, L)] = jnp.zeros((L,), jnp.int32)

    def scan_body(jj, carry):
        run, npos = carry
        j = (nb // L - 1) - jj
        tot = hist[pl.ds(j * L, L)]
        schunk = lax.rev(plsc.cumsum(lax.rev(tot, (0,))), (0,)) + run
        sref[pl.ds(j * L, L)] = schunk
        npos = npos + plsc.all_reduce_population_count(schunk >= r)
        return schunk[0], npos

    _, nposv = lax.fori_loop(
        0, nb // L, scan_body, (jnp.int32(0), jnp.zeros((L,), jnp.int32)))
    bstar = nposv[0] - 1
    s_above = plsc.load_gather(
        sref, [jnp.full((L,), bstar + 1, jnp.int32)])[0]
    return bstar, s_above


def _zero_hist(hist):
    for j in range(NB // L):
        hist[pl.ds(j * L, L)] = jnp.zeros((L,), jnp.int32)


def _row_quantile(data, cand, hist, sref):
    ABS_MASK = jnp.int32(ABS_MASK_I)
    lanes = lax.iota(jnp.int32, L)
    ones = jnp.ones((L,), jnp.int32)

    _zero_hist(hist)

    @plsc.parallel_loop(0, NV, unroll=UNROLL)
    def _(j):
        x = data[pl.ds(j * L, L)]
        b = lax.shift_right_logical(
            plsc.bitcast(x, jnp.int32), jnp.int32(23)) & jnp.int32(0xFF)
        plsc.addupdate_scatter(hist, [b], ones)

    b0, s_above = _scan_level(hist, sref, jnp.int32(RANK), NB)
    r = jnp.int32(RANK) - s_above

    t0 = b0 * jnp.int32(1 << 23)

    @plsc.parallel_loop(0, NV, unroll=UNROLL, carry=lanes)
    def off16(j, off):
        x = data[pl.ds(j * L, L)]
        u = plsc.bitcast(x, jnp.int32) & ABS_MASK
        m_ge = u >= t0
        plsc.store_scatter(cand, [off], u, mask=m_ge)
        return off + jnp.where(m_ge, jnp.int32(L), jnp.int32(0))

    off_vec = lax.shift_right_logical(off16 - lanes, jnp.int32(4))
    jmax = jnp.max(off_vec)

    prefix = b0
    for shift, width in ((15, 8), (7, 8), (0, 7)):
        nb = 1 << width
        _zero_hist(hist)

        @plsc.parallel_loop(0, jmax, unroll=2)
        def _(j, shift=shift, nb=nb, prefix=prefix, r=r):
            u = cand[pl.ds(j * L, L)]
            b = lax.shift_right_logical(u, jnp.int32(shift)) & jnp.int32(nb - 1)
            m = (off_vec > j) & (
                lax.shift_right_logical(u, jnp.int32(shift + width)) == prefix)
            plsc.addupdate_scatter(hist, [b], ones, mask=m)

        bl, s_above = _scan_level(hist, sref, r, nb)
        r = r - s_above
        prefix = prefix * jnp.int32(nb) + bl

    t = prefix

    @plsc.parallel_loop(0, jmax, unroll=2, carry=jnp.full((L,), ABS_MASK))
    def accv(j, acc):
        u = cand[pl.ds(j * L, L)]
        return jnp.minimum(acc, jnp.where((off_vec > j) & (u > t), u, ABS_MASK))

    v256b = jnp.where(r == jnp.int32(1), jnp.min(accv), t)

    v257f = plsc.bitcast(jnp.full((L,), t, jnp.int32), jnp.float32)
    v256f = plsc.bitcast(jnp.full((L,), v256b, jnp.int32), jnp.float32)
    return v257f + jnp.float32(0.0078125) * (v256f - v257f)


def _threshold_row(data, qv):
    ABS_MASK = jnp.int32(ABS_MASK_I)
    SIGN_MASK = jnp.int32(SIGN_MASK_I)

    @plsc.parallel_loop(0, NV, unroll=UNROLL)
    def _(j):
        x = data[pl.ds(j * L, L)]
        bits = plsc.bitcast(x, jnp.int32)
        af = plsc.bitcast(bits & ABS_MASK, jnp.float32)
        d = jnp.maximum(af - qv, jnp.float32(0.0))
        o = plsc.bitcast(d, jnp.int32) | (bits & SIGN_MASK)
        data[pl.ds(j * L, L)] = plsc.bitcast(o, jnp.float32)


def _sc_call(x):
    mesh = plsc.VectorSubcoreMesh(
        core_axis_name="c", subcore_axis_name="s",
        num_cores=NC, num_subcores=NS)

    @functools.partial(
        pl.kernel,
        out_type=jax.ShapeDtypeStruct((NUM_ROWS, N), jnp.float32),
        mesh=mesh,
        compiler_params=pltpu.CompilerParams(needs_layout_passes=False),
        scratch_types=[
            pltpu.VMEM((N,), jnp.float32),
            pltpu.VMEM((N,), jnp.float32),
            pltpu.VMEM((N,), jnp.int32),
            pltpu.VMEM((NB + L,), jnp.int32),
            pltpu.VMEM((NB + L,), jnp.int32),
            pltpu.SemaphoreType.DMA,
            pltpu.SemaphoreType.DMA,
            pltpu.SemaphoreType.DMA,
            pltpu.SemaphoreType.DMA,
        ],
    )
    def k(x_hbm, out_hbm, data0, data1, cand, hist, sref,
          lsem0, lsem1, ssem0, ssem1):
        wid = lax.axis_index("s") * NC + lax.axis_index("c")
        base = wid * ROWS_PER_W
        bufs = (data0, data1)
        lsems = (lsem0, lsem1)
        ssems = (ssem0, ssem1)

        loads = [pltpu.async_copy(x_hbm.at[base], data0, lsem0)]
        stores = [None, None]
        for i in range(ROWS_PER_W):
            cur = i % 2
            nxt = (i + 1) % 2
            if i + 1 < ROWS_PER_W:
                if stores[nxt] is not None:
                    stores[nxt].wait()
                    stores[nxt] = None
                loads.append(pltpu.async_copy(
                    x_hbm.at[base + i + 1], bufs[nxt], lsems[nxt]))
            loads[i].wait()
            data = bufs[cur]
            qv = _row_quantile(data, cand, hist, sref)
            _threshold_row(data, qv)
            stores[cur] = pltpu.async_copy(
                data, out_hbm.at[base + i], ssems[cur])
        for s in stores:
            if s is not None:
                s.wait()

    return k(x)


def kernel(X):
    return _sc_call(X)
